# Initial kernel scaffold; baseline (speedup 1.0000x reference)
import jax
import jax.numpy as jnp
from jax.experimental import pallas as pl


def kernel(x, edge_index, batch, W1, b1, Ws1, bs1, W2, b2, Ws2, bs2, W3, b3, Ws3, bs3, L1, bl1, L2, bl2, L3, bl3):
    raise NotImplementedError("write your pallas kernel here")


# speedup vs baseline: 10.2525x; 10.2525x over previous
import functools
import jax
import jax.numpy as jnp
from jax import lax
from jax.experimental import pallas as pl
from jax.experimental.pallas import tpu as pltpu
from jax.experimental.pallas import tpu_sc as plsc

N = 10000
E = 320000
G = 64
F = 128
NPAD = 10240
NBLK = E // 128
WIN = 512
NPW = NPAD // 16

_mesh = plsc.VectorSubcoreMesh(core_axis_name="c", subcore_axis_name="s")


@functools.partial(
    pl.kernel, mesh=_mesh,
    compiler_params=pltpu.CompilerParams(use_tc_tiling_on_sc=False, needs_layout_passes=False),
    out_type=jax.ShapeDtypeStruct((2 * NPAD,), jnp.float32),
    scratch_types=[
        pltpu.VMEM_SHARED((NPAD,), jnp.float32),
        pltpu.VMEM((NPAD // 16, 16), jnp.float32),
        pltpu.VMEM((NPAD // 16, 16), jnp.float32),
        pltpu.VMEM((128,), jnp.float32),
        pltpu.VMEM((1, 128), jnp.int32),
        pltpu.VMEM((1, 128), jnp.int32),
        pltpu.VMEM((NPW,), jnp.float32),
    ],
)
def _sc_scalar(src_hbm, a_hbm, row_hbm, col_hbm, out_hbm,
               acc_sh, src_v, a_v, val_v, rowb_v, colb_v, zb_v):
    c = lax.axis_index("c")
    s = lax.axis_index("s")
    wid = c * 16 + s

    pltpu.sync_copy(src_hbm, src_v)
    pltpu.sync_copy(a_hbm, a_v)

    def _z(i, carry):
        zb_v[pl.ds(i * 16, 16)] = jnp.zeros((16,), jnp.float32)
        return carry
    lax.fori_loop(0, NPW // 16, _z, 0)
    pltpu.sync_copy(zb_v, acc_sh.at[pl.ds(s * NPW, NPW)])
    plsc.subcore_barrier()

    nb = NBLK // 32
    rem = NBLK - nb * 32
    lo = wid * nb + jnp.minimum(wid, rem)
    cnt = nb + jnp.where(wid < rem, 1, 0)

    def _blk(t, carry):
        bt = lo + t
        pltpu.sync_copy(row_hbm.at[pl.ds(bt, 1)], rowb_v)
        pltpu.sync_copy(col_hbm.at[pl.ds(bt, 1)], colb_v)

        def _g(j, carry2):
            r16 = rowb_v[0, pl.ds(j * 16, 16)]
            c16 = colb_v[0, pl.ds(j * 16, 16)]
            rq, rr = r16 >> 4, r16 & 15
            cq, cr = c16 >> 4, c16 & 15
            v = (plsc.load_gather(src_v, [rq, rr])
                 * plsc.load_gather(a_v, [rq, rr])
                 * plsc.load_gather(a_v, [cq, cr]))
            val_v[pl.ds(j * 16, 16)] = v
            return carry2
        lax.fori_loop(0, 8, _g, 0)

        def _s(j, carry2):
            c16 = colb_v[0, pl.ds(j * 16, 16)]
            pltpu.sync_copy(val_v.at[pl.ds(j * 16, 16)],
                            acc_sh.at[c16], add=True)
            return carry2
        lax.fori_loop(0, 8, _s, 0)
        return carry
    lax.fori_loop(0, cnt, _blk, 0)

    plsc.subcore_barrier()
    pltpu.sync_copy(acc_sh.at[pl.ds(s * NPW, NPW)],
                    out_hbm.at[pl.ds(c * NPAD + s * NPW, NPW)])


@functools.partial(
    pl.kernel, mesh=_mesh,
    compiler_params=pltpu.CompilerParams(use_tc_tiling_on_sc=False, needs_layout_passes=False),
    out_type=jax.ShapeDtypeStruct((2 * NPAD, F), jnp.float32),
    scratch_types=[
        pltpu.VMEM_SHARED((NPAD, F), jnp.float32),
        pltpu.VMEM((NPAD // 16, 16), jnp.float32),
        pltpu.VMEM((128, F), jnp.float32),
        pltpu.VMEM((128,), jnp.float32),
        pltpu.VMEM((1, 128), jnp.int32),
        pltpu.VMEM((1, 128), jnp.int32),
        pltpu.VMEM((128, F), jnp.float32),
        pltpu.SemaphoreType.DMA,
    ],
)
def _sc_feat(h_hbm, dinv_hbm, row_hbm, col_hbm, out_hbm,
             acc_sh, dinv_v, rows_v, coef_v, rowb_v, colb_v, zb_v, sem):
    c = lax.axis_index("c")
    s = lax.axis_index("s")
    wid = c * 16 + s

    pltpu.sync_copy(dinv_hbm, dinv_v)

    def _z(i, carry):
        for f in range(8):
            zb_v[i, pl.ds(f * 16, 16)] = jnp.zeros((16,), jnp.float32)
        return carry
    lax.fori_loop(0, 128, _z, 0)
    for t in range(NPW // 128):
        pltpu.sync_copy(zb_v, acc_sh.at[pl.ds(s * NPW + t * 128, 128)])
    plsc.subcore_barrier()

    nb = NBLK // 32
    rem = NBLK - nb * 32
    lo = wid * nb + jnp.minimum(wid, rem)
    cnt = nb + jnp.where(wid < rem, 1, 0)

    def _blk(t, carry):
        bt = lo + t
        pltpu.sync_copy(row_hbm.at[pl.ds(bt, 1)], rowb_v)
        pltpu.sync_copy(col_hbm.at[pl.ds(bt, 1)], colb_v)

        def _gs(j, carry2):
            r16 = rowb_v[0, pl.ds(j * 16, 16)]
            c16 = colb_v[0, pl.ds(j * 16, 16)]
            rv16 = rows_v.at[pl.ds(j * 16, 16), :]
            pltpu.async_copy(h_hbm.at[r16], rv16, sem).wait()
            cfv = (plsc.load_gather(dinv_v, [r16 >> 4, r16 & 15])
                   * plsc.load_gather(dinv_v, [c16 >> 4, c16 & 15]))
            for l in range(16):
                i = j * 16 + l
                cf = cfv[l]
                for f in range(8):
                    rows_v[i, pl.ds(f * 16, 16)] = (
                        rows_v[i, pl.ds(f * 16, 16)] * cf)
            pltpu.sync_copy(rows_v.at[pl.ds(j * 16, 16), :],
                            acc_sh.at[c16], add=True)
            return carry2
        lax.fori_loop(0, 8, _gs, 0)
        return carry
    lax.fori_loop(0, cnt, _blk, 0)

    plsc.subcore_barrier()
    for t in range(NPW // 128):
        pltpu.sync_copy(
            acc_sh.at[pl.ds(s * NPW + t * 128, 128)],
            out_hbm.at[pl.ds(c * NPAD + s * NPW + t * 128, 128)])


def _mm_body(x_ref, w_ref, o_ref):
    o_ref[...] = jnp.dot(x_ref[...], w_ref[...],
                         preferred_element_type=jnp.float32)


_mm = pl.pallas_call(
    _mm_body,
    grid=(25,),
    in_specs=[pl.BlockSpec((400, F), lambda i: (i, 0)),
              pl.BlockSpec((F, F), lambda i: (0, 0))],
    out_specs=pl.BlockSpec((400, F), lambda i: (i, 0)),
    out_shape=jax.ShapeDtypeStruct((N, F), jnp.float32),
)


def _mv_body(x_ref, w_ref, o_ref):
    o_ref[...] = jnp.sum(x_ref[...] * w_ref[...].reshape(1, F),
                         axis=1, keepdims=True)


_mv = pl.pallas_call(
    _mv_body,
    grid=(25,),
    in_specs=[pl.BlockSpec((400, F), lambda i: (i, 0)),
              pl.BlockSpec((F, 1), lambda i: (0, 0))],
    out_specs=pl.BlockSpec((400, 1), lambda i: (i, 0)),
    out_shape=jax.ShapeDtypeStruct((N, 1), jnp.float32),
)


def _dinv_body(s_ref, nm_ref, o_ref):
    ssum = s_ref[0] + s_ref[1]
    deg = nm_ref[...] * (ssum + 1.0)
    o_ref[...] = jnp.where(deg > 0, lax.rsqrt(jnp.maximum(deg, 1e-30)), 0.0)


_dinv_k = pl.pallas_call(
    _dinv_body,
    grid=(25,),
    in_specs=[pl.BlockSpec((2, 400, 1), lambda i: (0, i, 0)),
              pl.BlockSpec((400, 1), lambda i: (i, 0))],
    out_specs=pl.BlockSpec((400, 1), lambda i: (i, 0)),
    out_shape=jax.ShapeDtypeStruct((N, 1), jnp.float32),
)


def _fin_body(acc_ref, hw_ref, dinv_ref, nm_ref, b_ref, o_ref, *, relu, width):
    d = dinv_ref[...]
    nmv = nm_ref[...]
    out = acc_ref[0] + acc_ref[1] + hw_ref[...] * (d * d * nmv)
    out = (out + b_ref[...]) * nmv
    if relu:
        out = jnp.maximum(out, 0.0)
    o_ref[...] = out


def _make_fin(relu, width):
    return pl.pallas_call(
        functools.partial(_fin_body, relu=relu, width=width),
        grid=(25,),
        in_specs=[pl.BlockSpec((2, 400, width), lambda i: (0, i, 0)),
                  pl.BlockSpec((400, width), lambda i: (i, 0)),
                  pl.BlockSpec((400, 1), lambda i: (i, 0)),
                  pl.BlockSpec((400, 1), lambda i: (i, 0)),
                  pl.BlockSpec((1, width), lambda i: (0, 0))],
        out_specs=pl.BlockSpec((400, width), lambda i: (i, 0)),
        out_shape=jax.ShapeDtypeStruct((N, width), jnp.float32),
    )


_fin_feat = _make_fin(True, F)
_fin_score = _make_fin(False, 1)


WINP = WIN + 8


def _pool_body(h_ref, s_ref, nm_ref, b_ref, ho_ref, nmo_ref, xg_ref):
    ii = lax.broadcasted_iota(jnp.int32, (WINP, WINP), 0)
    jj = lax.broadcasted_iota(jnp.int32, (WINP, WINP), 1)
    eye = (ii == jj).astype(jnp.float32)
    loc_i = lax.broadcasted_iota(jnp.int32, (WINP, 1), 0)

    def per_g(g, carry):
        bcol = b_ref[...]
        start = jnp.sum((bcol < g).astype(jnp.int32))
        slen = jnp.sum((bcol == g).astype(jnp.int32))
        al = (start // 8) * 8
        off = start - al
        lo = loc_i - off
        sw = s_ref[pl.ds(al, WINP), :]
        nmw = nm_ref[pl.ds(al, WINP), :]
        inseg = jnp.logical_and(lo >= 0, lo < slen)
        validc = inseg.astype(jnp.float32)
        acnt = jnp.sum(nmw * validc)
        k = jnp.ceil(0.5 * acnt).astype(jnp.int32)

        neg = jnp.float32(-3.0e38)
        seff_c = jnp.where(jnp.logical_and(nmw > 0, inseg), sw, neg)
        dn = (((0,), (0,)), ((), ()))
        seff_r = lax.dot_general(seff_c, eye, dn,
                                 preferred_element_type=jnp.float32)
        valid_r = lax.dot_general(validc, eye, dn,
                                  preferred_element_type=jnp.float32)
        gt = (seff_r > seff_c).astype(jnp.float32)
        tie = jnp.logical_and(seff_r == seff_c, jj < ii).astype(jnp.float32)
        rank = jnp.sum((gt + tie) * valid_r, axis=1, keepdims=True)
        keep = jnp.logical_and(rank < k.astype(jnp.float32), inseg)
        keep = jnp.logical_and(keep, nmw > 0)
        keepf = keep.astype(jnp.float32)

        gate = jnp.tanh(sw) * keepf
        hw = h_ref[pl.ds(al, WINP), :]
        hg = hw * gate
        insegf = validc
        ho_prev = ho_ref[pl.ds(al, WINP), :]
        nmo_prev = nmo_ref[pl.ds(al, WINP), :]
        ho_ref[pl.ds(al, WINP), :] = jnp.where(insegf > 0, hg, ho_prev)
        nmo_ref[pl.ds(al, WINP), :] = jnp.where(insegf > 0, keepf, nmo_prev)

        cntn = jnp.sum(keepf)
        mean = jnp.sum(hg, axis=0, keepdims=True) / jnp.maximum(cntn, 1.0)
        mx = jnp.max(jnp.where(keepf > 0, hg, neg), axis=0, keepdims=True)
        mx = jnp.where(mx > neg, mx, 0.0)
        row = jnp.concatenate([mx, mean], axis=1)
        xg_ref[pl.ds(g * 8, 8), :] = jnp.broadcast_to(row, (8, 2 * F))
        return carry

    lax.fori_loop(0, G, per_g, 0)


_pool = pl.pallas_call(
    _pool_body,
    out_shape=[jax.ShapeDtypeStruct((N + WINP, F), jnp.float32),
               jax.ShapeDtypeStruct((N + WINP, 1), jnp.float32),
               jax.ShapeDtypeStruct((G * 8, 2 * F), jnp.float32)],
)


def _head_body(x1, x2, x3, l1, b1, l2, b2, l3, b3, o_ref):
    z = x1[...] + x2[...] + x3[...]
    z = jnp.maximum(jnp.dot(z, l1[...], preferred_element_type=jnp.float32)
                    + b1[...], 0.0)
    z = jnp.maximum(jnp.dot(z, l2[...], preferred_element_type=jnp.float32)
                    + b2[...], 0.0)
    z = jnp.dot(z, l3[...], preferred_element_type=jnp.float32) + b3[...]
    m = jnp.max(z, axis=1, keepdims=True)
    lse = jnp.log(jnp.sum(jnp.exp(z - m), axis=1, keepdims=True)) + m
    o_ref[...] = z - lse


_head = pl.pallas_call(
    _head_body,
    out_shape=jax.ShapeDtypeStruct((G, F), jnp.float32),
)


def kernel(x, edge_index, batch, W1, b1, Ws1, bs1, W2, b2, Ws2, bs2,
           W3, b3, Ws3, bs3, L1, bl1, L2, bl2, L3, bl3):
    f32 = jnp.float32
    row2d = edge_index[0].reshape(NBLK, 128)
    col2d = edge_index[1].reshape(NBLK, 128)
    batch_c = jnp.concatenate(
        [batch, jnp.full((WINP,), G, jnp.int32)]).reshape(N + WINP, 1)

    nm = jnp.ones((N, 1), f32)
    ones_n = jnp.ones((NPAD // 16, 16), f32)
    ztail = jnp.zeros((NPAD - N,), f32)
    zpadF = jnp.zeros((WINP, F), f32)
    zpad1 = jnp.zeros((WINP, 1), f32)

    h = x
    xs = []
    for (Wm, b, Wsm, bs) in ((W1, b1, Ws1, bs1), (W2, b2, Ws2, bs2),
                             (W3, b3, Ws3, bs3)):
        sp = _sc_scalar(jnp.concatenate([nm[:, 0], ztail]).reshape(-1, 16),
                        ones_n, row2d, col2d)
        sp = sp.reshape(2, NPAD, 1)[:, :N]
        dinv = _dinv_k(sp, nm)
        dinv_p = jnp.concatenate([dinv[:, 0], ztail]).reshape(-1, 16)
        hW = _mm(h, Wm)
        fp = _sc_feat(hW, dinv_p, row2d, col2d)
        fp = fp.reshape(2, NPAD, F)[:, :N]
        h1 = _fin_feat(fp, hW, dinv, nm, b.reshape(1, F))
        hs = _mv(h1, Wsm.reshape(F, 1))
        ssp = _sc_scalar(jnp.concatenate([hs[:, 0], ztail]).reshape(-1, 16),
                         dinv_p, row2d, col2d)
        ssp = ssp.reshape(2, NPAD, 1)[:, :N]
        s = _fin_score(ssp, hs, dinv, nm, bs.reshape(1, 1))
        hp = jnp.concatenate([h1, zpadF], axis=0)
        spad = jnp.concatenate([s, zpad1], axis=0)
        nmp = jnp.concatenate([nm, zpad1], axis=0)
        ho, nmo, xg = _pool(hp, spad, nmp, batch_c)
        h = ho[:N]
        nm = nmo[:N]
        xs.append(xg[::8])

    l3p = jnp.pad(L3, ((0, 0), (0, F - 10)))
    bl3p = jnp.concatenate([bl3, jnp.full((F - 10,), -1e30, f32)])
    out = _head(xs[0], xs[1], xs[2], L1, bl1.reshape(1, F),
                L2, bl2.reshape(1, F // 2), l3p, bl3p.reshape(1, F))
    return out[:, :10]


# speedup vs baseline: 13.7004x; 1.3363x over previous
import functools
import jax
import jax.numpy as jnp
from jax import lax
from jax.experimental import pallas as pl
from jax.experimental.pallas import tpu as pltpu
from jax.experimental.pallas import tpu_sc as plsc

N = 10000
E = 320000
G = 64
F = 128
NPAD = 10240
NBLK = E // 128
WIN = 512
NPW = NPAD // 16

_mesh = plsc.VectorSubcoreMesh(core_axis_name="c", subcore_axis_name="s")


@functools.partial(
    pl.kernel, mesh=_mesh,
    compiler_params=pltpu.CompilerParams(use_tc_tiling_on_sc=False, needs_layout_passes=False),
    out_type=jax.ShapeDtypeStruct((2 * NPAD,), jnp.float32),
    scratch_types=[
        pltpu.VMEM_SHARED((NPAD,), jnp.float32),
        pltpu.VMEM((NPAD // 16, 16), jnp.float32),
        pltpu.VMEM((NPAD // 16, 16), jnp.float32),
        pltpu.VMEM((128,), jnp.float32),
        pltpu.VMEM((1, 128), jnp.int32),
        pltpu.VMEM((1, 128), jnp.int32),
        pltpu.VMEM((NPW,), jnp.float32),
    ],
)
def _sc_scalar(src_hbm, a_hbm, row_hbm, col_hbm, out_hbm,
               acc_sh, src_v, a_v, val_v, rowb_v, colb_v, zb_v):
    c = lax.axis_index("c")
    s = lax.axis_index("s")
    wid = c * 16 + s

    pltpu.sync_copy(src_hbm, src_v)
    pltpu.sync_copy(a_hbm, a_v)

    def _z(i, carry):
        zb_v[pl.ds(i * 16, 16)] = jnp.zeros((16,), jnp.float32)
        return carry
    lax.fori_loop(0, NPW // 16, _z, 0)
    pltpu.sync_copy(zb_v, acc_sh.at[pl.ds(s * NPW, NPW)])
    plsc.subcore_barrier()

    nb = NBLK // 32
    rem = NBLK - nb * 32
    lo = wid * nb + jnp.minimum(wid, rem)
    cnt = nb + jnp.where(wid < rem, 1, 0)

    def _blk(t, carry):
        bt = lo + t
        pltpu.sync_copy(row_hbm.at[pl.ds(bt, 1)], rowb_v)
        pltpu.sync_copy(col_hbm.at[pl.ds(bt, 1)], colb_v)

        def _g(j, carry2):
            r16 = rowb_v[0, pl.ds(j * 16, 16)]
            c16 = colb_v[0, pl.ds(j * 16, 16)]
            rq, rr = r16 >> 4, r16 & 15
            cq, cr = c16 >> 4, c16 & 15
            v = (plsc.load_gather(src_v, [rq, rr])
                 * plsc.load_gather(a_v, [rq, rr])
                 * plsc.load_gather(a_v, [cq, cr]))
            val_v[pl.ds(j * 16, 16)] = v
            return carry2
        lax.fori_loop(0, 8, _g, 0)

        def _s(j, carry2):
            c16 = colb_v[0, pl.ds(j * 16, 16)]
            pltpu.sync_copy(val_v.at[pl.ds(j * 16, 16)],
                            acc_sh.at[c16], add=True)
            return carry2
        lax.fori_loop(0, 8, _s, 0)
        return carry
    lax.fori_loop(0, cnt, _blk, 0)

    plsc.subcore_barrier()
    pltpu.sync_copy(acc_sh.at[pl.ds(s * NPW, NPW)],
                    out_hbm.at[pl.ds(c * NPAD + s * NPW, NPW)])


@functools.partial(
    pl.kernel, mesh=_mesh,
    compiler_params=pltpu.CompilerParams(use_tc_tiling_on_sc=False, needs_layout_passes=False),
    out_type=jax.ShapeDtypeStruct((2 * NPAD, F), jnp.float32),
    scratch_types=[
        pltpu.VMEM_SHARED((NPAD, F), jnp.float32),
        pltpu.VMEM((NPAD // 16, 16), jnp.float32),
        pltpu.VMEM((128, F), jnp.float32),
        pltpu.VMEM((128,), jnp.float32),
        pltpu.VMEM((1, 128), jnp.int32),
        pltpu.VMEM((1, 128), jnp.int32),
        pltpu.VMEM((128, F), jnp.float32),
        pltpu.SemaphoreType.DMA,
    ],
)
def _sc_feat(h_hbm, dinv_hbm, row_hbm, col_hbm, out_hbm,
             acc_sh, dinv_v, rows_v, coef_v, rowb_v, colb_v, zb_v, sem):
    c = lax.axis_index("c")
    s = lax.axis_index("s")
    wid = c * 16 + s

    pltpu.sync_copy(dinv_hbm, dinv_v)

    def _z(i, carry):
        for f in range(8):
            zb_v[i, pl.ds(f * 16, 16)] = jnp.zeros((16,), jnp.float32)
        return carry
    lax.fori_loop(0, 128, _z, 0)
    for t in range(NPW // 128):
        pltpu.sync_copy(zb_v, acc_sh.at[pl.ds(s * NPW + t * 128, 128)])
    plsc.subcore_barrier()

    nb = NBLK // 32
    rem = NBLK - nb * 32
    lo = wid * nb + jnp.minimum(wid, rem)
    cnt = nb + jnp.where(wid < rem, 1, 0)

    def _blk(t, carry):
        bt = lo + t
        pltpu.sync_copy(row_hbm.at[pl.ds(bt, 1)], rowb_v)
        pltpu.sync_copy(col_hbm.at[pl.ds(bt, 1)], colb_v)
        pltpu.async_copy(h_hbm.at[rowb_v.at[0]], rows_v, sem).wait()

        def _gs(j, carry2):
            r16 = rowb_v[0, pl.ds(j * 16, 16)]
            c16 = colb_v[0, pl.ds(j * 16, 16)]
            cfv = (plsc.load_gather(dinv_v, [r16 >> 4, r16 & 15])
                   * plsc.load_gather(dinv_v, [c16 >> 4, c16 & 15]))
            for l in range(16):
                i = j * 16 + l
                cf = cfv[l]
                for f in range(8):
                    rows_v[i, pl.ds(f * 16, 16)] = (
                        rows_v[i, pl.ds(f * 16, 16)] * cf)
            pltpu.sync_copy(rows_v.at[pl.ds(j * 16, 16), :],
                            acc_sh.at[c16], add=True)
            return carry2
        lax.fori_loop(0, 8, _gs, 0)
        return carry
    lax.fori_loop(0, cnt, _blk, 0)

    plsc.subcore_barrier()
    for t in range(NPW // 128):
        pltpu.sync_copy(
            acc_sh.at[pl.ds(s * NPW + t * 128, 128)],
            out_hbm.at[pl.ds(c * NPAD + s * NPW + t * 128, 128)])


def _mm_body(x_ref, w_ref, o_ref):
    o_ref[...] = jnp.dot(x_ref[...], w_ref[...],
                         preferred_element_type=jnp.float32)


_mm = pl.pallas_call(
    _mm_body,
    grid=(25,),
    in_specs=[pl.BlockSpec((400, F), lambda i: (i, 0)),
              pl.BlockSpec((F, F), lambda i: (0, 0))],
    out_specs=pl.BlockSpec((400, F), lambda i: (i, 0)),
    out_shape=jax.ShapeDtypeStruct((N, F), jnp.float32),
)


def _mv_body(x_ref, w_ref, o_ref):
    o_ref[...] = jnp.sum(x_ref[...] * w_ref[...].reshape(1, F),
                         axis=1, keepdims=True)


_mv = pl.pallas_call(
    _mv_body,
    grid=(25,),
    in_specs=[pl.BlockSpec((400, F), lambda i: (i, 0)),
              pl.BlockSpec((F, 1), lambda i: (0, 0))],
    out_specs=pl.BlockSpec((400, 1), lambda i: (i, 0)),
    out_shape=jax.ShapeDtypeStruct((N, 1), jnp.float32),
)


def _dinv_body(s_ref, nm_ref, o_ref):
    ssum = s_ref[0] + s_ref[1]
    deg = nm_ref[...] * (ssum + 1.0)
    o_ref[...] = jnp.where(deg > 0, lax.rsqrt(jnp.maximum(deg, 1e-30)), 0.0)


_dinv_k = pl.pallas_call(
    _dinv_body,
    grid=(25,),
    in_specs=[pl.BlockSpec((2, 400, 1), lambda i: (0, i, 0)),
              pl.BlockSpec((400, 1), lambda i: (i, 0))],
    out_specs=pl.BlockSpec((400, 1), lambda i: (i, 0)),
    out_shape=jax.ShapeDtypeStruct((N, 1), jnp.float32),
)


def _fin_body(acc_ref, hw_ref, dinv_ref, nm_ref, b_ref, o_ref, *, relu, width):
    d = dinv_ref[...]
    nmv = nm_ref[...]
    out = acc_ref[0] + acc_ref[1] + hw_ref[...] * (d * d * nmv)
    out = (out + b_ref[...]) * nmv
    if relu:
        out = jnp.maximum(out, 0.0)
    o_ref[...] = out


def _make_fin(relu, width):
    return pl.pallas_call(
        functools.partial(_fin_body, relu=relu, width=width),
        grid=(25,),
        in_specs=[pl.BlockSpec((2, 400, width), lambda i: (0, i, 0)),
                  pl.BlockSpec((400, width), lambda i: (i, 0)),
                  pl.BlockSpec((400, 1), lambda i: (i, 0)),
                  pl.BlockSpec((400, 1), lambda i: (i, 0)),
                  pl.BlockSpec((1, width), lambda i: (0, 0))],
        out_specs=pl.BlockSpec((400, width), lambda i: (i, 0)),
        out_shape=jax.ShapeDtypeStruct((N, width), jnp.float32),
    )


_fin_feat = _make_fin(True, F)
_fin_score = _make_fin(False, 1)


WINP = WIN + 8


def _pool_body(h_ref, s_ref, nm_ref, b_ref, ho_ref, nmo_ref, xg_ref):
    ii = lax.broadcasted_iota(jnp.int32, (WINP, WINP), 0)
    jj = lax.broadcasted_iota(jnp.int32, (WINP, WINP), 1)
    eye = (ii == jj).astype(jnp.float32)
    loc_i = lax.broadcasted_iota(jnp.int32, (WINP, 1), 0)

    def per_g(g, carry):
        bcol = b_ref[...]
        start = jnp.sum((bcol < g).astype(jnp.int32))
        slen = jnp.sum((bcol == g).astype(jnp.int32))
        al = (start // 8) * 8
        off = start - al
        lo = loc_i - off
        sw = s_ref[pl.ds(al, WINP), :]
        nmw = nm_ref[pl.ds(al, WINP), :]
        inseg = jnp.logical_and(lo >= 0, lo < slen)
        validc = inseg.astype(jnp.float32)
        acnt = jnp.sum(nmw * validc)
        k = jnp.ceil(0.5 * acnt).astype(jnp.int32)

        neg = jnp.float32(-3.0e38)
        seff_c = jnp.where(jnp.logical_and(nmw > 0, inseg), sw, neg)
        dn = (((0,), (0,)), ((), ()))
        seff_r = lax.dot_general(seff_c, eye, dn,
                                 preferred_element_type=jnp.float32)
        valid_r = lax.dot_general(validc, eye, dn,
                                  preferred_element_type=jnp.float32)
        gt = (seff_r > seff_c).astype(jnp.float32)
        tie = jnp.logical_and(seff_r == seff_c, jj < ii).astype(jnp.float32)
        rank = jnp.sum((gt + tie) * valid_r, axis=1, keepdims=True)
        keep = jnp.logical_and(rank < k.astype(jnp.float32), inseg)
        keep = jnp.logical_and(keep, nmw > 0)
        keepf = keep.astype(jnp.float32)

        gate = jnp.tanh(sw) * keepf
        hw = h_ref[pl.ds(al, WINP), :]
        hg = hw * gate
        insegf = validc
        ho_prev = ho_ref[pl.ds(al, WINP), :]
        nmo_prev = nmo_ref[pl.ds(al, WINP), :]
        ho_ref[pl.ds(al, WINP), :] = jnp.where(insegf > 0, hg, ho_prev)
        nmo_ref[pl.ds(al, WINP), :] = jnp.where(insegf > 0, keepf, nmo_prev)

        cntn = jnp.sum(keepf)
        mean = jnp.sum(hg, axis=0, keepdims=True) / jnp.maximum(cntn, 1.0)
        mx = jnp.max(jnp.where(keepf > 0, hg, neg), axis=0, keepdims=True)
        mx = jnp.where(mx > neg, mx, 0.0)
        row = jnp.concatenate([mx, mean], axis=1)
        xg_ref[pl.ds(g * 8, 8), :] = jnp.broadcast_to(row, (8, 2 * F))
        return carry

    lax.fori_loop(0, G, per_g, 0)


_pool = pl.pallas_call(
    _pool_body,
    out_shape=[jax.ShapeDtypeStruct((N + WINP, F), jnp.float32),
               jax.ShapeDtypeStruct((N + WINP, 1), jnp.float32),
               jax.ShapeDtypeStruct((G * 8, 2 * F), jnp.float32)],
)


def _head_body(x1, x2, x3, l1, b1, l2, b2, l3, b3, o_ref):
    z = x1[...] + x2[...] + x3[...]
    z = jnp.maximum(jnp.dot(z, l1[...], preferred_element_type=jnp.float32)
                    + b1[...], 0.0)
    z = jnp.maximum(jnp.dot(z, l2[...], preferred_element_type=jnp.float32)
                    + b2[...], 0.0)
    z = jnp.dot(z, l3[...], preferred_element_type=jnp.float32) + b3[...]
    m = jnp.max(z, axis=1, keepdims=True)
    lse = jnp.log(jnp.sum(jnp.exp(z - m), axis=1, keepdims=True)) + m
    o_ref[...] = z - lse


_head = pl.pallas_call(
    _head_body,
    out_shape=jax.ShapeDtypeStruct((G, F), jnp.float32),
)


def kernel(x, edge_index, batch, W1, b1, Ws1, bs1, W2, b2, Ws2, bs2,
           W3, b3, Ws3, bs3, L1, bl1, L2, bl2, L3, bl3):
    f32 = jnp.float32
    row2d = edge_index[0].reshape(NBLK, 128)
    col2d = edge_index[1].reshape(NBLK, 128)
    batch_c = jnp.concatenate(
        [batch, jnp.full((WINP,), G, jnp.int32)]).reshape(N + WINP, 1)

    nm = jnp.ones((N, 1), f32)
    ones_n = jnp.ones((NPAD // 16, 16), f32)
    ztail = jnp.zeros((NPAD - N,), f32)
    zpadF = jnp.zeros((WINP, F), f32)
    zpad1 = jnp.zeros((WINP, 1), f32)

    h = x
    xs = []
    for (Wm, b, Wsm, bs) in ((W1, b1, Ws1, bs1), (W2, b2, Ws2, bs2),
                             (W3, b3, Ws3, bs3)):
        sp = _sc_scalar(jnp.concatenate([nm[:, 0], ztail]).reshape(-1, 16),
                        ones_n, row2d, col2d)
        sp = sp.reshape(2, NPAD, 1)[:, :N]
        dinv = _dinv_k(sp, nm)
        dinv_p = jnp.concatenate([dinv[:, 0], ztail]).reshape(-1, 16)
        hW = _mm(h, Wm)
        fp = _sc_feat(hW, dinv_p, row2d, col2d)
        fp = fp.reshape(2, NPAD, F)[:, :N]
        h1 = _fin_feat(fp, hW, dinv, nm, b.reshape(1, F))
        hs = _mv(h1, Wsm.reshape(F, 1))
        ssp = _sc_scalar(jnp.concatenate([hs[:, 0], ztail]).reshape(-1, 16),
                         dinv_p, row2d, col2d)
        ssp = ssp.reshape(2, NPAD, 1)[:, :N]
        s = _fin_score(ssp, hs, dinv, nm, bs.reshape(1, 1))
        hp = jnp.concatenate([h1, zpadF], axis=0)
        spad = jnp.concatenate([s, zpad1], axis=0)
        nmp = jnp.concatenate([nm, zpad1], axis=0)
        ho, nmo, xg = _pool(hp, spad, nmp, batch_c)
        h = ho[:N]
        nm = nmo[:N]
        xs.append(xg[::8])

    l3p = jnp.pad(L3, ((0, 0), (0, F - 10)))
    bl3p = jnp.concatenate([bl3, jnp.full((F - 10,), -1e30, f32)])
    out = _head(xs[0], xs[1], xs[2], L1, bl1.reshape(1, F),
                L2, bl2.reshape(1, F // 2), l3p, bl3p.reshape(1, F))
    return out[:, :10]


# speedup vs baseline: 15.1827x; 1.1082x over previous
import functools
import jax
import jax.numpy as jnp
from jax import lax
from jax.experimental import pallas as pl
from jax.experimental.pallas import tpu as pltpu
from jax.experimental.pallas import tpu_sc as plsc

N = 10000
E = 320000
G = 64
F = 128
NPAD = 10240
NBLK = E // 128
WIN = 512
NPW = NPAD // 16

_mesh = plsc.VectorSubcoreMesh(core_axis_name="c", subcore_axis_name="s")


@functools.partial(
    pl.kernel, mesh=_mesh,
    compiler_params=pltpu.CompilerParams(use_tc_tiling_on_sc=False, needs_layout_passes=False),
    out_type=jax.ShapeDtypeStruct((2 * NPAD,), jnp.float32),
    scratch_types=[
        pltpu.VMEM_SHARED((NPAD,), jnp.float32),
        pltpu.VMEM((NPAD // 16, 16), jnp.float32),
        pltpu.VMEM((NPAD // 16, 16), jnp.float32),
        pltpu.VMEM((128,), jnp.float32),
        pltpu.VMEM((1, 128), jnp.int32),
        pltpu.VMEM((1, 128), jnp.int32),
        pltpu.VMEM((NPW,), jnp.float32),
    ],
)
def _sc_scalar(src_hbm, a_hbm, row_hbm, col_hbm, out_hbm,
               acc_sh, src_v, a_v, val_v, rowb_v, colb_v, zb_v):
    c = lax.axis_index("c")
    s = lax.axis_index("s")
    wid = c * 16 + s

    pltpu.sync_copy(src_hbm, src_v)
    pltpu.sync_copy(a_hbm, a_v)

    def _z(i, carry):
        zb_v[pl.ds(i * 16, 16)] = jnp.zeros((16,), jnp.float32)
        return carry
    lax.fori_loop(0, NPW // 16, _z, 0)
    pltpu.sync_copy(zb_v, acc_sh.at[pl.ds(s * NPW, NPW)])
    plsc.subcore_barrier()

    nb = NBLK // 32
    rem = NBLK - nb * 32
    lo = wid * nb + jnp.minimum(wid, rem)
    cnt = nb + jnp.where(wid < rem, 1, 0)

    def _blk(t, carry):
        bt = lo + t
        pltpu.sync_copy(row_hbm.at[pl.ds(bt, 1)], rowb_v)
        pltpu.sync_copy(col_hbm.at[pl.ds(bt, 1)], colb_v)

        def _g(j, carry2):
            r16 = rowb_v[0, pl.ds(j * 16, 16)]
            c16 = colb_v[0, pl.ds(j * 16, 16)]
            rq, rr = r16 >> 4, r16 & 15
            cq, cr = c16 >> 4, c16 & 15
            v = (plsc.load_gather(src_v, [rq, rr])
                 * plsc.load_gather(a_v, [rq, rr])
                 * plsc.load_gather(a_v, [cq, cr]))
            val_v[pl.ds(j * 16, 16)] = v
            return carry2
        lax.fori_loop(0, 8, _g, 0)

        pltpu.sync_copy(val_v, acc_sh.at[colb_v.at[0]], add=True)
        return carry
    lax.fori_loop(0, cnt, _blk, 0)

    plsc.subcore_barrier()
    pltpu.sync_copy(acc_sh.at[pl.ds(s * NPW, NPW)],
                    out_hbm.at[pl.ds(c * NPAD + s * NPW, NPW)])


@functools.partial(
    pl.kernel, mesh=_mesh,
    compiler_params=pltpu.CompilerParams(use_tc_tiling_on_sc=False, needs_layout_passes=False),
    out_type=jax.ShapeDtypeStruct((2 * NPAD, F), jnp.float32),
    scratch_types=[
        pltpu.VMEM_SHARED((NPAD, F), jnp.float32),
        pltpu.VMEM((NPAD // 16, 16), jnp.float32),
        pltpu.VMEM((128, F), jnp.float32),
        pltpu.VMEM((128,), jnp.float32),
        pltpu.VMEM((1, 128), jnp.int32),
        pltpu.VMEM((1, 128), jnp.int32),
        pltpu.VMEM((128, F), jnp.float32),
        pltpu.SemaphoreType.DMA,
    ],
)
def _sc_feat(h_hbm, dinv_hbm, row_hbm, col_hbm, out_hbm,
             acc_sh, dinv_v, rows_v, coef_v, rowb_v, colb_v, zb_v, sem):
    c = lax.axis_index("c")
    s = lax.axis_index("s")
    wid = c * 16 + s

    pltpu.sync_copy(dinv_hbm, dinv_v)

    def _z(i, carry):
        for f in range(8):
            zb_v[i, pl.ds(f * 16, 16)] = jnp.zeros((16,), jnp.float32)
        return carry
    lax.fori_loop(0, 128, _z, 0)
    for t in range(NPW // 128):
        pltpu.sync_copy(zb_v, acc_sh.at[pl.ds(s * NPW + t * 128, 128)])
    plsc.subcore_barrier()

    nb = NBLK // 32
    rem = NBLK - nb * 32
    lo = wid * nb + jnp.minimum(wid, rem)
    cnt = nb + jnp.where(wid < rem, 1, 0)

    def _blk(t, carry):
        bt = lo + t
        pltpu.sync_copy(row_hbm.at[pl.ds(bt, 1)], rowb_v)
        pltpu.sync_copy(col_hbm.at[pl.ds(bt, 1)], colb_v)
        pltpu.async_copy(h_hbm.at[rowb_v.at[0]], rows_v, sem).wait()

        def _gs(j, carry2):
            r16 = rowb_v[0, pl.ds(j * 16, 16)]
            c16 = colb_v[0, pl.ds(j * 16, 16)]
            cfv = (plsc.load_gather(dinv_v, [r16 >> 4, r16 & 15])
                   * plsc.load_gather(dinv_v, [c16 >> 4, c16 & 15]))
            for l in range(16):
                i = j * 16 + l
                cf = cfv[l]
                for f in range(8):
                    rows_v[i, pl.ds(f * 16, 16)] = (
                        rows_v[i, pl.ds(f * 16, 16)] * cf)
            return carry2
        lax.fori_loop(0, 8, _gs, 0)
        pltpu.sync_copy(rows_v, acc_sh.at[colb_v.at[0]], add=True)
        return carry
    lax.fori_loop(0, cnt, _blk, 0)

    plsc.subcore_barrier()
    for t in range(NPW // 128):
        pltpu.sync_copy(
            acc_sh.at[pl.ds(s * NPW + t * 128, 128)],
            out_hbm.at[pl.ds(c * NPAD + s * NPW + t * 128, 128)])


def _mm_body(x_ref, w_ref, o_ref):
    o_ref[...] = jnp.dot(x_ref[...], w_ref[...],
                         preferred_element_type=jnp.float32)


_mm = pl.pallas_call(
    _mm_body,
    grid=(25,),
    in_specs=[pl.BlockSpec((400, F), lambda i: (i, 0)),
              pl.BlockSpec((F, F), lambda i: (0, 0))],
    out_specs=pl.BlockSpec((400, F), lambda i: (i, 0)),
    out_shape=jax.ShapeDtypeStruct((N, F), jnp.float32),
)


def _mv_body(x_ref, w_ref, o_ref):
    o_ref[...] = jnp.sum(x_ref[...] * w_ref[...].reshape(1, F),
                         axis=1, keepdims=True)


_mv = pl.pallas_call(
    _mv_body,
    grid=(25,),
    in_specs=[pl.BlockSpec((400, F), lambda i: (i, 0)),
              pl.BlockSpec((F, 1), lambda i: (0, 0))],
    out_specs=pl.BlockSpec((400, 1), lambda i: (i, 0)),
    out_shape=jax.ShapeDtypeStruct((N, 1), jnp.float32),
)


def _dinv_body(s_ref, nm_ref, o_ref):
    ssum = s_ref[0] + s_ref[1]
    deg = nm_ref[...] * (ssum + 1.0)
    o_ref[...] = jnp.where(deg > 0, lax.rsqrt(jnp.maximum(deg, 1e-30)), 0.0)


_dinv_k = pl.pallas_call(
    _dinv_body,
    grid=(25,),
    in_specs=[pl.BlockSpec((2, 400, 1), lambda i: (0, i, 0)),
              pl.BlockSpec((400, 1), lambda i: (i, 0))],
    out_specs=pl.BlockSpec((400, 1), lambda i: (i, 0)),
    out_shape=jax.ShapeDtypeStruct((N, 1), jnp.float32),
)


def _fin_body(acc_ref, hw_ref, dinv_ref, nm_ref, b_ref, o_ref, *, relu, width):
    d = dinv_ref[...]
    nmv = nm_ref[...]
    out = acc_ref[0] + acc_ref[1] + hw_ref[...] * (d * d * nmv)
    out = (out + b_ref[...]) * nmv
    if relu:
        out = jnp.maximum(out, 0.0)
    o_ref[...] = out


def _make_fin(relu, width):
    return pl.pallas_call(
        functools.partial(_fin_body, relu=relu, width=width),
        grid=(25,),
        in_specs=[pl.BlockSpec((2, 400, width), lambda i: (0, i, 0)),
                  pl.BlockSpec((400, width), lambda i: (i, 0)),
                  pl.BlockSpec((400, 1), lambda i: (i, 0)),
                  pl.BlockSpec((400, 1), lambda i: (i, 0)),
                  pl.BlockSpec((1, width), lambda i: (0, 0))],
        out_specs=pl.BlockSpec((400, width), lambda i: (i, 0)),
        out_shape=jax.ShapeDtypeStruct((N, width), jnp.float32),
    )


_fin_feat = _make_fin(True, F)
_fin_score = _make_fin(False, 1)


WINP = WIN + 8


def _pool_body(h_ref, s_ref, nm_ref, b_ref, ho_ref, nmo_ref, xg_ref):
    ii = lax.broadcasted_iota(jnp.int32, (WINP, WINP), 0)
    jj = lax.broadcasted_iota(jnp.int32, (WINP, WINP), 1)
    eye = (ii == jj).astype(jnp.float32)
    loc_i = lax.broadcasted_iota(jnp.int32, (WINP, 1), 0)

    def per_g(g, carry):
        bcol = b_ref[...]
        start = jnp.sum((bcol < g).astype(jnp.int32))
        slen = jnp.sum((bcol == g).astype(jnp.int32))
        al = (start // 8) * 8
        off = start - al
        lo = loc_i - off
        sw = s_ref[pl.ds(al, WINP), :]
        nmw = nm_ref[pl.ds(al, WINP), :]
        inseg = jnp.logical_and(lo >= 0, lo < slen)
        validc = inseg.astype(jnp.float32)
        acnt = jnp.sum(nmw * validc)
        k = jnp.ceil(0.5 * acnt).astype(jnp.int32)

        neg = jnp.float32(-3.0e38)
        seff_c = jnp.where(jnp.logical_and(nmw > 0, inseg), sw, neg)
        dn = (((0,), (0,)), ((), ()))
        seff_r = lax.dot_general(seff_c, eye, dn,
                                 preferred_element_type=jnp.float32)
        valid_r = lax.dot_general(validc, eye, dn,
                                  preferred_element_type=jnp.float32)
        gt = (seff_r > seff_c).astype(jnp.float32)
        tie = jnp.logical_and(seff_r == seff_c, jj < ii).astype(jnp.float32)
        rank = jnp.sum((gt + tie) * valid_r, axis=1, keepdims=True)
        keep = jnp.logical_and(rank < k.astype(jnp.float32), inseg)
        keep = jnp.logical_and(keep, nmw > 0)
        keepf = keep.astype(jnp.float32)

        gate = jnp.tanh(sw) * keepf
        hw = h_ref[pl.ds(al, WINP), :]
        hg = hw * gate
        insegf = validc
        ho_prev = ho_ref[pl.ds(al, WINP), :]
        nmo_prev = nmo_ref[pl.ds(al, WINP), :]
        ho_ref[pl.ds(al, WINP), :] = jnp.where(insegf > 0, hg, ho_prev)
        nmo_ref[pl.ds(al, WINP), :] = jnp.where(insegf > 0, keepf, nmo_prev)

        cntn = jnp.sum(keepf)
        mean = jnp.sum(hg, axis=0, keepdims=True) / jnp.maximum(cntn, 1.0)
        mx = jnp.max(jnp.where(keepf > 0, hg, neg), axis=0, keepdims=True)
        mx = jnp.where(mx > neg, mx, 0.0)
        row = jnp.concatenate([mx, mean], axis=1)
        xg_ref[pl.ds(g * 8, 8), :] = jnp.broadcast_to(row, (8, 2 * F))
        return carry

    lax.fori_loop(0, G, per_g, 0)


_pool = pl.pallas_call(
    _pool_body,
    out_shape=[jax.ShapeDtypeStruct((N + WINP, F), jnp.float32),
               jax.ShapeDtypeStruct((N + WINP, 1), jnp.float32),
               jax.ShapeDtypeStruct((G * 8, 2 * F), jnp.float32)],
)


def _head_body(x1, x2, x3, l1, b1, l2, b2, l3, b3, o_ref):
    z = x1[...] + x2[...] + x3[...]
    z = jnp.maximum(jnp.dot(z, l1[...], preferred_element_type=jnp.float32)
                    + b1[...], 0.0)
    z = jnp.maximum(jnp.dot(z, l2[...], preferred_element_type=jnp.float32)
                    + b2[...], 0.0)
    z = jnp.dot(z, l3[...], preferred_element_type=jnp.float32) + b3[...]
    m = jnp.max(z, axis=1, keepdims=True)
    lse = jnp.log(jnp.sum(jnp.exp(z - m), axis=1, keepdims=True)) + m
    o_ref[...] = z - lse


_head = pl.pallas_call(
    _head_body,
    out_shape=jax.ShapeDtypeStruct((G, F), jnp.float32),
)


def kernel(x, edge_index, batch, W1, b1, Ws1, bs1, W2, b2, Ws2, bs2,
           W3, b3, Ws3, bs3, L1, bl1, L2, bl2, L3, bl3):
    f32 = jnp.float32
    row2d = edge_index[0].reshape(NBLK, 128)
    col2d = edge_index[1].reshape(NBLK, 128)
    batch_c = jnp.concatenate(
        [batch, jnp.full((WINP,), G, jnp.int32)]).reshape(N + WINP, 1)

    nm = jnp.ones((N, 1), f32)
    ones_n = jnp.ones((NPAD // 16, 16), f32)
    ztail = jnp.zeros((NPAD - N,), f32)
    zpadF = jnp.zeros((WINP, F), f32)
    zpad1 = jnp.zeros((WINP, 1), f32)

    h = x
    xs = []
    for (Wm, b, Wsm, bs) in ((W1, b1, Ws1, bs1), (W2, b2, Ws2, bs2),
                             (W3, b3, Ws3, bs3)):
        sp = _sc_scalar(jnp.concatenate([nm[:, 0], ztail]).reshape(-1, 16),
                        ones_n, row2d, col2d)
        sp = sp.reshape(2, NPAD, 1)[:, :N]
        dinv = _dinv_k(sp, nm)
        dinv_p = jnp.concatenate([dinv[:, 0], ztail]).reshape(-1, 16)
        hW = _mm(h, Wm)
        fp = _sc_feat(hW, dinv_p, row2d, col2d)
        fp = fp.reshape(2, NPAD, F)[:, :N]
        h1 = _fin_feat(fp, hW, dinv, nm, b.reshape(1, F))
        hs = _mv(h1, Wsm.reshape(F, 1))
        ssp = _sc_scalar(jnp.concatenate([hs[:, 0], ztail]).reshape(-1, 16),
                         dinv_p, row2d, col2d)
        ssp = ssp.reshape(2, NPAD, 1)[:, :N]
        s = _fin_score(ssp, hs, dinv, nm, bs.reshape(1, 1))
        hp = jnp.concatenate([h1, zpadF], axis=0)
        spad = jnp.concatenate([s, zpad1], axis=0)
        nmp = jnp.concatenate([nm, zpad1], axis=0)
        ho, nmo, xg = _pool(hp, spad, nmp, batch_c)
        h = ho[:N]
        nm = nmo[:N]
        xs.append(xg[::8])

    l3p = jnp.pad(L3, ((0, 0), (0, F - 10)))
    bl3p = jnp.concatenate([bl3, jnp.full((F - 10,), -1e30, f32)])
    out = _head(xs[0], xs[1], xs[2], L1, bl1.reshape(1, F),
                L2, bl2.reshape(1, F // 2), l3p, bl3p.reshape(1, F))
    return out[:, :10]
